# Initial kernel scaffold; baseline (speedup 1.0000x reference)
import functools

import jax
import jax.numpy as jnp
from jax import lax
from jax.experimental import pallas as pl
from jax.experimental.pallas import tpu as pltpu
from jax.experimental.pallas import tpu_sc as plsc

_D = 128
_NC = 2
_NS = 16
_NW = _NC * _NS


def _gather_kernel(num_ids, chunk):
    n_chunks = num_ids // (_NW * chunk)
    b_per_w = num_ids // _NW
    mesh = plsc.VectorSubcoreMesh(core_axis_name="c", subcore_axis_name="s")

    @functools.partial(
        pl.kernel,
        mesh=mesh,
        out_type=jax.ShapeDtypeStruct((num_ids, _D), jnp.float32),
        scratch_types=[
            pltpu.VMEM((2, chunk), jnp.int32),
            pltpu.VMEM((2, chunk, _D), jnp.float32),
            pltpu.SemaphoreType.DMA,
        ],
    )
    def body(idx_hbm, table_hbm, out_hbm, idx_v, rows_v, sem):
        wid = lax.axis_index("s") * _NC + lax.axis_index("c")
        base = wid * b_per_w

        def step(i, _):
            slot = lax.rem(i, 2)
            off = base + i * chunk
            pltpu.sync_copy(idx_hbm.at[pl.ds(off, chunk)], idx_v.at[slot])
            pltpu.async_copy(table_hbm.at[idx_v.at[slot]], rows_v.at[slot], sem).wait()
            pltpu.sync_copy(rows_v.at[slot], out_hbm.at[pl.ds(off, chunk)])
            return ()

        lax.fori_loop(0, n_chunks, step, (), unroll=False)

    return body


def kernel(token_ids, weight):
    b, s = token_ids.shape
    flat = token_ids.reshape(-1).astype(jnp.int32)
    out = _gather_kernel(b * s, 400)(flat, weight)
    return out.reshape(b, s, _D)


# speedup vs baseline: 3.1729x; 3.1729x over previous
import functools

import jax
import jax.numpy as jnp
from jax import lax
from jax.experimental import pallas as pl
from jax.experimental.pallas import tpu as pltpu
from jax.experimental.pallas import tpu_sc as plsc

_D = 128
_NC = 2
_NS = 16
_NW = _NC * _NS


def _gather_kernel(num_ids, chunk):
    n_chunks = num_ids // (_NW * chunk)
    b_per_w = num_ids // _NW
    mesh = plsc.VectorSubcoreMesh(core_axis_name="c", subcore_axis_name="s")

    @functools.partial(
        pl.kernel,
        mesh=mesh,
        out_type=jax.ShapeDtypeStruct((num_ids, _D), jnp.float32),
        scratch_types=[
            pltpu.VMEM((chunk,), jnp.int32),
            pltpu.VMEM((chunk, _D), jnp.float32),
            pltpu.SemaphoreType.DMA,
        ],
    )
    def body(idx_hbm, table_hbm, out_hbm, idx_v, rows_v, sem):
        wid = lax.axis_index("s") * _NC + lax.axis_index("c")
        base = wid * b_per_w

        def step(i, _):
            off = base + i * chunk
            pltpu.sync_copy(idx_hbm.at[pl.ds(off, chunk)], idx_v)
            pltpu.async_copy(table_hbm.at[idx_v], rows_v, sem).wait()
            pltpu.sync_copy(rows_v, out_hbm.at[pl.ds(off, chunk)])
            return ()

        lax.fori_loop(0, n_chunks, step, (), unroll=False)

    return body


def kernel(token_ids, weight):
    b, s = token_ids.shape
    flat = token_ids.reshape(-1).astype(jnp.int32)
    out = _gather_kernel(b * s, 400)(flat, weight)
    return out.reshape(b, s, _D)


# speedup vs baseline: 3.3058x; 1.0419x over previous
import functools

import jax
import jax.numpy as jnp
from jax import lax
from jax.experimental import pallas as pl
from jax.experimental.pallas import tpu as pltpu
from jax.experimental.pallas import tpu_sc as plsc

_D = 128
_NC = 2
_NS = 16
_NW = _NC * _NS


def _gather_kernel(num_ids, chunk):
    b_per_w = num_ids // _NW
    n_chunks = b_per_w // chunk
    mesh = plsc.VectorSubcoreMesh(core_axis_name="c", subcore_axis_name="s")

    @functools.partial(
        pl.kernel,
        mesh=mesh,
        out_type=jax.ShapeDtypeStruct((num_ids, _D), jnp.float32),
        scratch_types=[
            pltpu.VMEM((b_per_w,), jnp.int32),
            pltpu.VMEM((chunk, _D), jnp.float32),
            pltpu.VMEM((chunk, _D), jnp.float32),
            pltpu.SemaphoreType.DMA,
            pltpu.SemaphoreType.DMA,
            pltpu.SemaphoreType.DMA,
            pltpu.SemaphoreType.DMA,
        ],
    )
    def body(idx_hbm, table_hbm, out_hbm, idx_v, rows_a, rows_b,
             gsem_a, gsem_b, osem_a, osem_b):
        wid = lax.axis_index("s") * _NC + lax.axis_index("c")
        base = wid * b_per_w
        rows = (rows_a, rows_b)
        gsem = (gsem_a, gsem_b)
        osem = (osem_a, osem_b)

        pltpu.sync_copy(idx_hbm.at[pl.ds(base, b_per_w)], idx_v)

        def gather(i, s):
            return pltpu.async_copy(
                table_hbm.at[idx_v.at[pl.ds(i * chunk, chunk)]], rows[s], gsem[s])

        def writeback(i, s):
            return pltpu.async_copy(
                rows[s], out_hbm.at[pl.ds(base + i * chunk, chunk)], osem[s])

        g0 = gather(0, 0)
        pending_g = {0: g0}
        pending_o = {}
        for i in range(n_chunks):
            s = i % 2
            pending_g.pop(s).wait()
            if i + 1 < n_chunks:
                s2 = (i + 1) % 2
                if s2 in pending_o:
                    pending_o.pop(s2).wait()
                pending_g[s2] = gather(i + 1, s2)
            pending_o[s] = writeback(i, s)
        for o in pending_o.values():
            o.wait()

    return body


def kernel(token_ids, weight):
    b, s = token_ids.shape
    flat = token_ids.reshape(-1).astype(jnp.int32)
    out = _gather_kernel(b * s, 400)(flat, weight)
    return out.reshape(b, s, _D)
